# baseline (speedup 1.0000x reference)
import functools

import jax
import jax.numpy as jnp
from jax import lax
from jax.experimental import pallas as pl
from jax.experimental.pallas import tpu as pltpu
from jax.experimental.pallas import tpu_sc as plsc

N = 10000
E = 320000
NFEAT = 128
HD = 64
NCLASS = 40
NLAYERS = 4

NC = 2
NS = 16
NTILES = NC * NS

NP = 10240
CH = 128
NCHUNK = 165
EPT = NCHUNK * CH
EP = NS * EPT
ESL = E + N
PAD_DST = 10100
RPT = NP // NS
HH = HD // NC
SWAVES = 8

_HIGHEST = jax.lax.Precision.HIGHEST


def _dot(a, b):
    return jnp.dot(a, b, preferred_element_type=jnp.float32, precision=_HIGHEST)


def _proj(h, w_ref, a2_ref, hw_ref, asad_ref, mm_ref, i):
    hw = _dot(h, w_ref[...])
    hw_ref[...] = hw
    asad = _dot(hw, a2_ref[...])
    asad_ref[...] = asad
    bm = jnp.max(asad, axis=0, keepdims=True)

    @pl.when(i == 0)
    def _():
        mm_ref[...] = bm

    @pl.when(i > 0)
    def _():
        mm_ref[...] = jnp.maximum(mm_ref[...], bm)


def _tc_init_body(x_ref, wi_ref, bi_ref, w_ref, a2_ref,
                  h_ref, hw_ref, asad_ref, mm_ref):
    i = pl.program_id(0)
    h = _dot(x_ref[...], wi_ref[...]) + bi_ref[...]
    h_ref[...] = h
    _proj(h, w_ref, a2_ref, hw_ref, asad_ref, mm_ref, i)


def _combine(acc0_ref, acc1_ref, den_ref, hin_ref, bl_ref):
    t = jnp.concatenate([acc0_ref[...], acc1_ref[...]], axis=1)
    d = den_ref[...] + 1e-16
    o = t / d + bl_ref[...]
    elu = jnp.where(o > 0.0, o, jnp.exp(jnp.minimum(o, 0.0)) - 1.0)
    return hin_ref[...] + elu


def _tc_mid_body(acc0_ref, acc1_ref, den_ref, hin_ref, bl_ref,
                 w_ref, a2_ref, h_ref, hw_ref, asad_ref, mm_ref):
    i = pl.program_id(0)
    g = _combine(acc0_ref, acc1_ref, den_ref, hin_ref, bl_ref)
    h_ref[...] = g
    _proj(g, w_ref, a2_ref, hw_ref, asad_ref, mm_ref, i)


def _tc_final_body(acc0_ref, acc1_ref, den_ref, hin_ref, bl_ref,
                   wo_ref, bo_ref, out_ref):
    g = _combine(acc0_ref, acc1_ref, den_ref, hin_ref, bl_ref)
    out_ref[...] = _dot(g, wo_ref[...]) + bo_ref[...]


_R = NP
_GRID = NP // _R


def _rows(width):
    return pl.BlockSpec((_R, width), lambda i: (i, 0))


def _whole(shape):
    return pl.BlockSpec(shape, lambda i: (0,) * len(shape))


def _tc_init(xp, wi, bi, w, a2):
    return pl.pallas_call(
        _tc_init_body,
        grid=(_GRID,),
        in_specs=[_rows(NFEAT), _whole((NFEAT, HD)), _whole((1, HD)),
                  _whole((HD, HD)), _whole((HD, 2))],
        out_specs=[_rows(HD), _rows(HD), _rows(2), _whole((1, 2))],
        out_shape=[jax.ShapeDtypeStruct((NP, HD), jnp.float32),
                   jax.ShapeDtypeStruct((NP, HD), jnp.float32),
                   jax.ShapeDtypeStruct((NP, 2), jnp.float32),
                   jax.ShapeDtypeStruct((1, 2), jnp.float32)],
    )(xp, wi, bi, w, a2)


def _tc_mid(acc0, acc1, den, hin, bl, w, a2):
    return pl.pallas_call(
        _tc_mid_body,
        grid=(_GRID,),
        in_specs=[_rows(HH), _rows(HH), _rows(1), _rows(HD),
                  _whole((1, HD)), _whole((HD, HD)), _whole((HD, 2))],
        out_specs=[_rows(HD), _rows(HD), _rows(2), _whole((1, 2))],
        out_shape=[jax.ShapeDtypeStruct((NP, HD), jnp.float32),
                   jax.ShapeDtypeStruct((NP, HD), jnp.float32),
                   jax.ShapeDtypeStruct((NP, 2), jnp.float32),
                   jax.ShapeDtypeStruct((1, 2), jnp.float32)],
    )(acc0, acc1, den, hin, bl, w, a2)


def _tc_final(acc0, acc1, den, hin, bl, wo, bo):
    return pl.pallas_call(
        _tc_final_body,
        grid=(_GRID,),
        in_specs=[_rows(HH), _rows(HH), _rows(1), _rows(HD),
                  _whole((1, HD)), _whole((HD, NCLASS)), _whole((1, NCLASS))],
        out_specs=[_rows(NCLASS)],
        out_shape=[jax.ShapeDtypeStruct((NP, NCLASS), jnp.float32)],
    )(acc0, acc1, den, hin, bl, wo, bo)


def _sc_body(src_r, dst_r, asad_r, mvec_r, hw_r,
             acc_out, den_out,
             src_v, dst_v, asad_v, mvec_v, pbuf,
             rows0, rows1, rows2, den_v, red_v, den_m,
             acc_s, stage_s, hw_s,
             gsem, ssem):
    c = lax.axis_index("c")
    s = lax.axis_index("s")

    pltpu.sync_copy(src_r.at[s], src_v)
    pltpu.sync_copy(dst_r.at[s], dst_v)
    pltpu.sync_copy(asad_r, asad_v)
    pltpu.sync_copy(mvec_r, mvec_v)
    mv = mvec_v[...]
    zv = jnp.zeros((16,), jnp.float32)

    def _z0(r, _):
        for cc in range(HH // 16):
            rows0[r, pl.ds(cc * 16, 16)] = zv
        return 0
    lax.fori_loop(0, CH, _z0, 0)

    def _z1(i, _):
        den_v[pl.ds(i * 16, 16)] = zv
        return 0
    lax.fori_loop(0, NP // 16, _z1, 0)

    for k in range(RPT // CH):
        sl = pl.ds(s * RPT + k * CH, CH)
        pltpu.sync_copy(rows0, acc_s.at[sl])
        pltpu.sync_copy(hw_r.at[sl, pl.ds(c * HH, HH)], rows1)
        pltpu.sync_copy(rows1, hw_s.at[sl])
    plsc.subcore_barrier()

    def _p_chunk(j):
        for k in range(CH // 16):
            sl = pl.ds(k * 16, 16)
            sv = src_v[j, sl]
            dv = dst_v[j, sl]
            av = plsc.load_gather(asad_v, [sv * 2])
            bv = plsc.load_gather(asad_v, [dv * 2 + 1])
            sm = av + bv
            e = jnp.maximum(sm, 0.2 * sm)
            q = mv + bv
            t = jnp.maximum(q, 0.2 * q)
            p = jnp.exp(e - t)
            pbuf[sl] = p
            plsc.addupdate_scatter(den_v, [dv], p)

    def _scale(rows):
        @plsc.parallel_loop(0, CH // 16, unroll=2)
        def _(k):
            pv = pbuf[pl.ds(k * 16, 16)]
            for r16 in range(16):
                pr = pv[r16]
                r = k * 16 + r16
                for cc in range(HH // 16):
                    sl = pl.ds(cc * 16, 16)
                    rows[r, sl] = rows[r, sl] * pr

    bufs = (rows0, rows1, rows2)
    pltpu.async_copy(hw_s.at[src_v.at[0]], rows0, gsem)

    def _triple(g, _):
        for b in range(3):
            rows, nxt = bufs[b], bufs[(b + 1) % 3]
            j = g * 3 + b
            _p_chunk(j)

            @pl.when(j >= 2)
            def _():
                pltpu.make_async_copy(
                    rows, acc_s.at[dst_v.at[j]], ssem).wait()

            pltpu.make_async_copy(hw_s.at[src_v.at[j]], rows, gsem).wait()

            @pl.when(j < NCHUNK - 1)
            def _():
                pltpu.async_copy(hw_s.at[src_v.at[j + 1]], nxt, gsem)

            _scale(rows)
            pltpu.async_copy(rows, acc_s.at[dst_v.at[j]], ssem, add=True)
        return 0

    lax.fori_loop(0, NCHUNK // 3, _triple, 0)
    for _ in range(2):
        pltpu.make_async_copy(
            rows0, acc_s.at[dst_v.at[NCHUNK - 1]], ssem).wait()
    plsc.subcore_barrier()

    def _red(i, _):
        sl = pl.ds(i * 16, 16)
        acc = red_v[0, sl]
        for r in range(1, NS):
            acc = acc + red_v[r, sl]
        den_m[sl] = acc
        return 0

    for w2 in range(SWAVES):
        pltpu.sync_copy(den_v.at[pl.ds(w2 * (NP // SWAVES), NP // SWAVES)],
                        stage_s.at[s])
        plsc.subcore_barrier()
        pltpu.sync_copy(stage_s.at[:, pl.ds(s * (RPT // SWAVES), RPT // SWAVES)],
                        red_v)
        lax.fori_loop(0, RPT // SWAVES // 16, _red, 0)
        pltpu.sync_copy(
            den_m,
            den_out.at[c, pl.ds(w2 * (NP // SWAVES) + s * (RPT // SWAVES),
                                RPT // SWAVES)])
        plsc.subcore_barrier()

    for k in range(RPT // CH):
        sl = pl.ds(s * RPT + k * CH, CH)
        pltpu.sync_copy(acc_s.at[sl], rows0)
        pltpu.sync_copy(rows0, acc_out.at[c, sl])


_sc_edge = pl.kernel(
    _sc_body,
    out_type=[jax.ShapeDtypeStruct((NC, NP, HH), jnp.float32),
              jax.ShapeDtypeStruct((NC, NP), jnp.float32)],
    mesh=plsc.VectorSubcoreMesh(core_axis_name="c", subcore_axis_name="s",
                                num_cores=NC, num_subcores=NS),
    scratch_types=[
        pltpu.VMEM((NCHUNK, CH), jnp.int32),
        pltpu.VMEM((NCHUNK, CH), jnp.int32),
        pltpu.VMEM((2 * NP,), jnp.float32),
        pltpu.VMEM((16,), jnp.float32),
        pltpu.VMEM((CH,), jnp.float32),
        pltpu.VMEM((CH, HH), jnp.float32),
        pltpu.VMEM((CH, HH), jnp.float32),
        pltpu.VMEM((CH, HH), jnp.float32),
        pltpu.VMEM((NP,), jnp.float32),
        pltpu.VMEM((NS, RPT // SWAVES), jnp.float32),
        pltpu.VMEM((RPT // SWAVES,), jnp.float32),
        pltpu.VMEM_SHARED((NP, HH), jnp.float32),
        pltpu.VMEM_SHARED((NS, NP // SWAVES), jnp.float32),
        pltpu.VMEM_SHARED((NP, HH), jnp.float32),
        pltpu.SemaphoreType.DMA,
        pltpu.SemaphoreType.DMA,
    ],
    compiler_params=pltpu.CompilerParams(needs_layout_passes=False,
                                         use_tc_tiling_on_sc=False),
)


def kernel(x, edge_index, lnin_w, lnin_b, conv_w, conv_att_src, conv_att_dst,
           conv_b, lnout_w, lnout_b):
    f32 = jnp.float32
    i32 = jnp.int32

    xp = jnp.zeros((NP, NFEAT), f32).at[:N].set(x)
    loop = jnp.arange(N, dtype=i32)
    src = jnp.concatenate([edge_index[0], loop,
                           jnp.zeros((EP - ESL,), i32)]).reshape(NS, NCHUNK, CH)
    dst = jnp.concatenate([edge_index[1], loop,
                           jnp.full((EP - ESL,), PAD_DST, i32)]).reshape(NS, NCHUNK, CH)

    def a2(l):
        return jnp.stack([conv_att_src[l], conv_att_dst[l]], axis=1)

    h, hw, asad, mm = _tc_init(xp, lnin_w, lnin_b.reshape(1, HD),
                               conv_w[0], a2(0))
    for l in range(NLAYERS):
        mvec = jnp.broadcast_to(mm[0, 0], (16,)).astype(f32)
        acc, den = _sc_edge(src, dst, asad.reshape(2 * NP), mvec, hw)
        den0 = den[0].reshape(NP, 1)
        bl = conv_b[l].reshape(1, HD)
        if l < NLAYERS - 1:
            h, hw, asad, mm = _tc_mid(acc[0], acc[1], den0, h, bl,
                                      conv_w[l + 1], a2(l + 1))
        else:
            (out,) = _tc_final(acc[0], acc[1], den0, h, bl,
                               lnout_w, lnout_b.reshape(1, NCLASS))
    return out[:N]


# speedup vs baseline: 1.0123x; 1.0123x over previous
import functools

import jax
import jax.numpy as jnp
from jax import lax
from jax.experimental import pallas as pl
from jax.experimental.pallas import tpu as pltpu
from jax.experimental.pallas import tpu_sc as plsc

N = 10000
E = 320000
NFEAT = 128
HD = 64
NCLASS = 40
NLAYERS = 4

NC = 2
NS = 16
NTILES = NC * NS

NP = 10240
CH = 128
NCHUNK = 165
EPT = NCHUNK * CH
EP = NS * EPT
ESL = E + N
PAD_DST = 10100
RPT = NP // NS
HH = HD // NC
SWAVES = 8

_HIGHEST = jax.lax.Precision.HIGHEST


def _dot(a, b):
    return jnp.dot(a, b, preferred_element_type=jnp.float32, precision=_HIGHEST)


def _proj(h, w_ref, a2_ref, hw_ref, asad_ref, mm_ref, i):
    hw = _dot(h, w_ref[...])
    hw_ref[...] = hw
    asad = _dot(hw, a2_ref[...])
    asad_ref[...] = asad
    bm = jnp.max(asad, axis=0, keepdims=True)

    @pl.when(i == 0)
    def _():
        mm_ref[...] = bm

    @pl.when(i > 0)
    def _():
        mm_ref[...] = jnp.maximum(mm_ref[...], bm)


def _tc_init_body(x_ref, wi_ref, bi_ref, w_ref, a2_ref,
                  h_ref, hw_ref, asad_ref, mm_ref):
    i = pl.program_id(0)
    h = _dot(x_ref[...], wi_ref[...]) + bi_ref[...]
    h_ref[...] = h
    _proj(h, w_ref, a2_ref, hw_ref, asad_ref, mm_ref, i)


def _combine(acc0_ref, acc1_ref, den_ref, hin_ref, bl_ref):
    t = jnp.concatenate([acc0_ref[...], acc1_ref[...]], axis=1)
    d = den_ref[...] + 1e-16
    o = t / d + bl_ref[...]
    elu = jnp.where(o > 0.0, o, jnp.exp(jnp.minimum(o, 0.0)) - 1.0)
    return hin_ref[...] + elu


def _tc_mid_body(acc0_ref, acc1_ref, den_ref, hin_ref, bl_ref,
                 w_ref, a2_ref, h_ref, hw_ref, asad_ref, mm_ref):
    i = pl.program_id(0)
    g = _combine(acc0_ref, acc1_ref, den_ref, hin_ref, bl_ref)
    h_ref[...] = g
    _proj(g, w_ref, a2_ref, hw_ref, asad_ref, mm_ref, i)


def _tc_final_body(acc0_ref, acc1_ref, den_ref, hin_ref, bl_ref,
                   wo_ref, bo_ref, out_ref):
    g = _combine(acc0_ref, acc1_ref, den_ref, hin_ref, bl_ref)
    out_ref[...] = _dot(g, wo_ref[...]) + bo_ref[...]


_R = 1024
_GRID = NP // _R


def _rows(width):
    return pl.BlockSpec((_R, width), lambda i: (i, 0))


def _whole(shape):
    return pl.BlockSpec(shape, lambda i: (0,) * len(shape))


def _tc_init(xp, wi, bi, w, a2):
    return pl.pallas_call(
        _tc_init_body,
        grid=(_GRID,),
        in_specs=[_rows(NFEAT), _whole((NFEAT, HD)), _whole((1, HD)),
                  _whole((HD, HD)), _whole((HD, 2))],
        out_specs=[_rows(HD), _rows(HD), _rows(2), _whole((1, 2))],
        out_shape=[jax.ShapeDtypeStruct((NP, HD), jnp.float32),
                   jax.ShapeDtypeStruct((NP, HD), jnp.float32),
                   jax.ShapeDtypeStruct((NP, 2), jnp.float32),
                   jax.ShapeDtypeStruct((1, 2), jnp.float32)],
    )(xp, wi, bi, w, a2)


def _tc_mid(acc0, acc1, den, hin, bl, w, a2):
    return pl.pallas_call(
        _tc_mid_body,
        grid=(_GRID,),
        in_specs=[_rows(HH), _rows(HH), _rows(1), _rows(HD),
                  _whole((1, HD)), _whole((HD, HD)), _whole((HD, 2))],
        out_specs=[_rows(HD), _rows(HD), _rows(2), _whole((1, 2))],
        out_shape=[jax.ShapeDtypeStruct((NP, HD), jnp.float32),
                   jax.ShapeDtypeStruct((NP, HD), jnp.float32),
                   jax.ShapeDtypeStruct((NP, 2), jnp.float32),
                   jax.ShapeDtypeStruct((1, 2), jnp.float32)],
    )(acc0, acc1, den, hin, bl, w, a2)


def _tc_final(acc0, acc1, den, hin, bl, wo, bo):
    return pl.pallas_call(
        _tc_final_body,
        grid=(_GRID,),
        in_specs=[_rows(HH), _rows(HH), _rows(1), _rows(HD),
                  _whole((1, HD)), _whole((HD, NCLASS)), _whole((1, NCLASS))],
        out_specs=[_rows(NCLASS)],
        out_shape=[jax.ShapeDtypeStruct((NP, NCLASS), jnp.float32)],
    )(acc0, acc1, den, hin, bl, wo, bo)


def _sc_body(src_r, dst_r, asad_r, mvec_r, hw_r,
             acc_out, den_out,
             src_v, dst_v, asad_v, mvec_v, pbuf,
             rows0, rows1, rows2, den_v, red_v, den_m,
             acc_s, stage_s, hw_s,
             gsem, ssem):
    c = lax.axis_index("c")
    s = lax.axis_index("s")

    pltpu.sync_copy(src_r.at[s], src_v)
    pltpu.sync_copy(dst_r.at[s], dst_v)
    pltpu.sync_copy(asad_r, asad_v)
    pltpu.sync_copy(mvec_r, mvec_v)
    mv = mvec_v[...]
    zv = jnp.zeros((16,), jnp.float32)

    def _z0(r, _):
        for cc in range(HH // 16):
            rows0[r, pl.ds(cc * 16, 16)] = zv
        return 0
    lax.fori_loop(0, CH, _z0, 0)

    def _z1(i, _):
        den_v[pl.ds(i * 16, 16)] = zv
        return 0
    lax.fori_loop(0, NP // 16, _z1, 0)

    for k in range(RPT // CH):
        sl = pl.ds(s * RPT + k * CH, CH)
        pltpu.sync_copy(rows0, acc_s.at[sl])
        pltpu.sync_copy(hw_r.at[sl, pl.ds(c * HH, HH)], rows1)
        pltpu.sync_copy(rows1, hw_s.at[sl])
    plsc.subcore_barrier()

    def _p_chunk(j):
        for k in range(CH // 16):
            sl = pl.ds(k * 16, 16)
            sv = src_v[j, sl]
            dv = dst_v[j, sl]
            av = plsc.load_gather(asad_v, [sv * 2])
            bv = plsc.load_gather(asad_v, [dv * 2 + 1])
            sm = av + bv
            e = jnp.maximum(sm, 0.2 * sm)
            q = mv + bv
            t = jnp.maximum(q, 0.2 * q)
            p = jnp.exp(e - t)
            pbuf[sl] = p
            plsc.addupdate_scatter(den_v, [dv], p)

    def _scale(rows):
        @plsc.parallel_loop(0, CH // 16, unroll=2)
        def _(k):
            pv = pbuf[pl.ds(k * 16, 16)]
            for r16 in range(16):
                pr = pv[r16]
                r = k * 16 + r16
                for cc in range(HH // 16):
                    sl = pl.ds(cc * 16, 16)
                    rows[r, sl] = rows[r, sl] * pr

    bufs = (rows0, rows1, rows2)
    pltpu.async_copy(hw_s.at[src_v.at[0]], rows0, gsem)

    def _triple(g, _):
        for b in range(3):
            rows, nxt = bufs[b], bufs[(b + 1) % 3]
            j = g * 3 + b
            _p_chunk(j)

            @pl.when(j >= 2)
            def _():
                pltpu.make_async_copy(
                    rows, acc_s.at[dst_v.at[j]], ssem).wait()

            pltpu.make_async_copy(hw_s.at[src_v.at[j]], rows, gsem).wait()

            @pl.when(j < NCHUNK - 1)
            def _():
                pltpu.async_copy(hw_s.at[src_v.at[j + 1]], nxt, gsem)

            _scale(rows)
            pltpu.async_copy(rows, acc_s.at[dst_v.at[j]], ssem, add=True)
        return 0

    lax.fori_loop(0, NCHUNK // 3, _triple, 0)
    for _ in range(2):
        pltpu.make_async_copy(
            rows0, acc_s.at[dst_v.at[NCHUNK - 1]], ssem).wait()
    plsc.subcore_barrier()

    def _red(i, _):
        sl = pl.ds(i * 16, 16)
        acc = red_v[0, sl]
        for r in range(1, NS):
            acc = acc + red_v[r, sl]
        den_m[sl] = acc
        return 0

    for w2 in range(SWAVES):
        pltpu.sync_copy(den_v.at[pl.ds(w2 * (NP // SWAVES), NP // SWAVES)],
                        stage_s.at[s])
        plsc.subcore_barrier()
        pltpu.sync_copy(stage_s.at[:, pl.ds(s * (RPT // SWAVES), RPT // SWAVES)],
                        red_v)
        lax.fori_loop(0, RPT // SWAVES // 16, _red, 0)
        pltpu.sync_copy(
            den_m,
            den_out.at[c, pl.ds(w2 * (NP // SWAVES) + s * (RPT // SWAVES),
                                RPT // SWAVES)])
        plsc.subcore_barrier()

    for k in range(RPT // CH):
        sl = pl.ds(s * RPT + k * CH, CH)
        pltpu.sync_copy(acc_s.at[sl], rows0)
        pltpu.sync_copy(rows0, acc_out.at[c, sl])


_sc_edge = pl.kernel(
    _sc_body,
    out_type=[jax.ShapeDtypeStruct((NC, NP, HH), jnp.float32),
              jax.ShapeDtypeStruct((NC, NP), jnp.float32)],
    mesh=plsc.VectorSubcoreMesh(core_axis_name="c", subcore_axis_name="s",
                                num_cores=NC, num_subcores=NS),
    scratch_types=[
        pltpu.VMEM((NCHUNK, CH), jnp.int32),
        pltpu.VMEM((NCHUNK, CH), jnp.int32),
        pltpu.VMEM((2 * NP,), jnp.float32),
        pltpu.VMEM((16,), jnp.float32),
        pltpu.VMEM((CH,), jnp.float32),
        pltpu.VMEM((CH, HH), jnp.float32),
        pltpu.VMEM((CH, HH), jnp.float32),
        pltpu.VMEM((CH, HH), jnp.float32),
        pltpu.VMEM((NP,), jnp.float32),
        pltpu.VMEM((NS, RPT // SWAVES), jnp.float32),
        pltpu.VMEM((RPT // SWAVES,), jnp.float32),
        pltpu.VMEM_SHARED((NP, HH), jnp.float32),
        pltpu.VMEM_SHARED((NS, NP // SWAVES), jnp.float32),
        pltpu.VMEM_SHARED((NP, HH), jnp.float32),
        pltpu.SemaphoreType.DMA,
        pltpu.SemaphoreType.DMA,
    ],
    compiler_params=pltpu.CompilerParams(needs_layout_passes=False,
                                         use_tc_tiling_on_sc=False),
)


def kernel(x, edge_index, lnin_w, lnin_b, conv_w, conv_att_src, conv_att_dst,
           conv_b, lnout_w, lnout_b):
    f32 = jnp.float32
    i32 = jnp.int32

    xp = jnp.zeros((NP, NFEAT), f32).at[:N].set(x)
    loop = jnp.arange(N, dtype=i32)
    src = jnp.concatenate([edge_index[0], loop,
                           jnp.zeros((EP - ESL,), i32)]).reshape(NS, NCHUNK, CH)
    dst = jnp.concatenate([edge_index[1], loop,
                           jnp.full((EP - ESL,), PAD_DST, i32)]).reshape(NS, NCHUNK, CH)

    def a2(l):
        return jnp.stack([conv_att_src[l], conv_att_dst[l]], axis=1)

    h, hw, asad, mm = _tc_init(xp, lnin_w, lnin_b.reshape(1, HD),
                               conv_w[0], a2(0))
    for l in range(NLAYERS):
        mvec = jnp.broadcast_to(mm[0, 0], (16,)).astype(f32)
        acc, den = _sc_edge(src, dst, asad.reshape(2 * NP), mvec, hw)
        den0 = den[0].reshape(NP, 1)
        bl = conv_b[l].reshape(1, HD)
        if l < NLAYERS - 1:
            h, hw, asad, mm = _tc_mid(acc[0], acc[1], den0, h, bl,
                                      conv_w[l + 1], a2(l + 1))
        else:
            (out,) = _tc_final(acc[0], acc[1], den0, h, bl,
                               lnout_w, lnout_b.reshape(1, NCLASS))
    return out[:N]
